# baseline (speedup 1.0000x reference)
import functools

import jax
import jax.numpy as jnp
from jax import lax
from jax.experimental import pallas as pl
from jax.experimental.pallas import tpu as pltpu
from jax.experimental.pallas import tpu_sc as plsc


def _gather_call(val, idx, num_cores, num_subcores, chunk):
    B, D = val.shape
    NW = num_cores * num_subcores
    b_per_w = B // NW
    n_ch = b_per_w // chunk

    mesh = plsc.VectorSubcoreMesh(core_axis_name="c", subcore_axis_name="s")

    @functools.partial(
        pl.kernel,
        mesh=mesh,
        out_type=jax.ShapeDtypeStruct((B, D), jnp.float32),
        compiler_params=pltpu.CompilerParams(use_tc_tiling_on_sc=False),
        scratch_types=[
            pltpu.VMEM((n_ch, chunk), jnp.int32),
            pltpu.VMEM((b_per_w, D), jnp.float32),
            pltpu.SemaphoreType.DMA,
            pltpu.SemaphoreType.DMA,
        ],
    )
    def gather_kernel(val_hbm, idx_hbm, out_hbm, idx_v, rows_v, gsem, osem):
        wid = lax.axis_index("s") * num_cores + lax.axis_index("c")
        base = wid * b_per_w
        pltpu.sync_copy(idx_hbm.at[wid], idx_v)
        gathers = [
            pltpu.async_copy(
                val_hbm.at[idx_v.at[j]],
                rows_v.at[pl.ds(j * chunk, chunk)],
                gsem,
            )
            for j in range(n_ch)
        ]
        outs = []
        for j in range(n_ch):
            gathers[j].wait()
            outs.append(
                pltpu.async_copy(
                    rows_v.at[pl.ds(j * chunk, chunk)],
                    out_hbm.at[pl.ds(base + j * chunk, chunk)],
                    osem,
                )
            )
        for c in outs:
            c.wait()

    return gather_kernel(val, idx.reshape(NW, n_ch, chunk))


def kernel(mem, val, indices):
    del mem
    info = plsc.get_sparse_core_info()
    idx = indices.astype(jnp.int32)
    return _gather_call(val, idx, info.num_cores, info.num_subcores, 128)


# speedup vs baseline: 1.1864x; 1.1864x over previous
import functools

import jax
import jax.numpy as jnp
from jax import lax
from jax.experimental import pallas as pl
from jax.experimental.pallas import tpu as pltpu
from jax.experimental.pallas import tpu_sc as plsc


def _gather_t_call(w, idx, num_cores, num_subcores, lanes):
    D, B = w.shape
    NW = num_cores * num_subcores
    rows_per_w = D // NW
    n_vec = B // lanes

    mesh = plsc.VectorSubcoreMesh(core_axis_name="c", subcore_axis_name="s")

    @functools.partial(
        pl.kernel,
        mesh=mesh,
        out_type=jax.ShapeDtypeStruct((D, B), jnp.float32),
        compiler_params=pltpu.CompilerParams(needs_layout_passes=False),
        scratch_types=[
            pltpu.VMEM((B,), jnp.int32),
            pltpu.VMEM((rows_per_w, B), jnp.float32),
            pltpu.VMEM((rows_per_w, B), jnp.float32),
        ],
    )
    def gather_kernel(w_hbm, idx_hbm, out_hbm, idx_v, rows_v, out_v):
        wid = lax.axis_index("s") * num_cores + lax.axis_index("c")
        r0 = wid * rows_per_w
        pltpu.sync_copy(idx_hbm, idx_v)
        pltpu.sync_copy(w_hbm.at[pl.ds(r0, rows_per_w)], rows_v)

        @pl.loop(0, n_vec)
        def body(k):
            col = pl.ds(k * lanes, lanes)
            iv = idx_v[col]
            for r in range(rows_per_w):
                rv = jnp.full((lanes,), r, dtype=jnp.int32)
                out_v[r, col] = plsc.load_gather(rows_v, [rv, iv])

        pltpu.sync_copy(out_v, out_hbm.at[pl.ds(r0, rows_per_w)])

    return gather_kernel(w, idx)


def kernel(mem, val, indices):
    del mem
    info = plsc.get_sparse_core_info()
    idx = indices.astype(jnp.int32)
    out_t = _gather_t_call(
        val.T, idx, info.num_cores, info.num_subcores, info.num_lanes
    )
    return out_t.T


# speedup vs baseline: 1.7728x; 1.4944x over previous
import functools

import jax
import jax.numpy as jnp
from jax import lax
from jax.experimental import pallas as pl
from jax.experimental.pallas import tpu as pltpu
from jax.experimental.pallas import tpu_sc as plsc


def _gather_t_call(w, idx, num_cores, num_subcores, lanes):
    D, B = w.shape
    NW = num_cores * num_subcores
    rows_per_w = D // NW
    n_vec = B // lanes

    mesh = plsc.VectorSubcoreMesh(core_axis_name="c", subcore_axis_name="s")

    @functools.partial(
        pl.kernel,
        mesh=mesh,
        out_type=jax.ShapeDtypeStruct((D, B), jnp.float32),
        compiler_params=pltpu.CompilerParams(needs_layout_passes=False),
        scratch_types=[
            pltpu.VMEM((B,), jnp.int32),
            pltpu.VMEM((rows_per_w, B), jnp.float32),
            pltpu.VMEM((rows_per_w, B), jnp.float32),
        ],
    )
    def gather_kernel(w_hbm, idx_hbm, out_hbm, idx_v, rows_v, out_v):
        wid = lax.axis_index("s") * num_cores + lax.axis_index("c")
        r0 = wid * rows_per_w
        pltpu.sync_copy(idx_hbm, idx_v)
        pltpu.sync_copy(w_hbm.at[pl.ds(r0, rows_per_w)], rows_v)

        @plsc.parallel_loop(0, n_vec, unroll=8)
        def body(k):
            col = pl.ds(k * lanes, lanes)
            iv = idx_v[col]
            for r in range(rows_per_w):
                rv = jnp.full((lanes,), r, dtype=jnp.int32)
                out_v[r, col] = plsc.load_gather(rows_v, [rv, iv])

        pltpu.sync_copy(out_v, out_hbm.at[pl.ds(r0, rows_per_w)])

    return gather_kernel(w, idx)


def kernel(mem, val, indices):
    del mem
    info = plsc.get_sparse_core_info()
    idx = indices.astype(jnp.int32)
    out_t = _gather_t_call(
        val.T, idx, info.num_cores, info.num_subcores, info.num_lanes
    )
    return out_t.T


# speedup vs baseline: 1.8295x; 1.0320x over previous
import functools

import jax
import jax.numpy as jnp
from jax import lax
from jax.experimental import pallas as pl
from jax.experimental.pallas import tpu as pltpu
from jax.experimental.pallas import tpu_sc as plsc


def _gather_t_call(w, idx, num_cores, num_subcores, lanes):
    D, B = w.shape
    NW = num_cores * num_subcores
    rows_per_w = D // NW
    n_vec = B // lanes

    mesh = plsc.VectorSubcoreMesh(core_axis_name="c", subcore_axis_name="s")

    @functools.partial(
        pl.kernel,
        mesh=mesh,
        out_type=jax.ShapeDtypeStruct((D, B), jnp.float32),
        compiler_params=pltpu.CompilerParams(needs_layout_passes=False),
        scratch_types=[
            pltpu.VMEM((B,), jnp.int32),
            pltpu.VMEM((rows_per_w, B), jnp.float32),
            pltpu.VMEM((rows_per_w, B), jnp.float32),
            pltpu.SemaphoreType.DMA,
            pltpu.SemaphoreType.DMA,
            pltpu.SemaphoreType.DMA,
        ],
    )
    def gather_kernel(w_hbm, idx_hbm, out_hbm, idx_v, rows_v, out_v,
                      isem, rsem, osem):
        wid = lax.axis_index("s") * num_cores + lax.axis_index("c")
        r0 = wid * rows_per_w
        ic = pltpu.async_copy(idx_hbm, idx_v, isem)
        rc = pltpu.async_copy(w_hbm.at[pl.ds(r0, rows_per_w)], rows_v, rsem)
        ic.wait()
        rc.wait()

        n_chunks = 4
        vec_per_chunk = n_vec // n_chunks
        outs = []
        for c in range(n_chunks):
            @plsc.parallel_loop(
                c * vec_per_chunk, (c + 1) * vec_per_chunk, unroll=8
            )
            def body(k):
                col = pl.ds(k * lanes, lanes)
                iv = idx_v[col]
                for r in range(rows_per_w):
                    rv = jnp.full((lanes,), r, dtype=jnp.int32)
                    out_v[r, col] = plsc.load_gather(rows_v, [rv, iv])

            cols = pl.ds(c * vec_per_chunk * lanes, vec_per_chunk * lanes)
            outs.append(
                pltpu.async_copy(
                    out_v.at[:, cols],
                    out_hbm.at[pl.ds(r0, rows_per_w), cols],
                    osem,
                )
            )
        for oc in outs:
            oc.wait()

    return gather_kernel(w, idx)


def kernel(mem, val, indices):
    del mem
    info = plsc.get_sparse_core_info()
    idx = indices.astype(jnp.int32)
    out_t = _gather_t_call(
        val.T, idx, info.num_cores, info.num_subcores, info.num_lanes
    )
    return out_t.T
